# baseline (speedup 1.0000x reference)
import jax
import jax.numpy as jnp
from jax.experimental import pallas as pl
from jax.experimental.pallas import tpu as pltpu

B = 1024
FT_IN = 41600
K_HALF = 256
NK = 65
BK = FT_IN // NK

_DN = (((1,), (1,)), ((), ()))


def _nnue_body(x1_ref, x2_ref, w_ref, ftb_ref, h1w_ref, h1b_ref,
               h2w_ref, h2b_ref, outw_ref, outb_ref, o_ref, acc1, acc2):
    k = pl.program_id(0)

    @pl.when(k == 0)
    def _init():
        acc1[...] = jnp.zeros_like(acc1)
        acc2[...] = jnp.zeros_like(acc2)

    w = w_ref[...]
    acc1[...] += jax.lax.dot_general(x1_ref[...], w, _DN,
                                     preferred_element_type=jnp.float32)
    acc2[...] += jax.lax.dot_general(x2_ref[...], w, _DN,
                                     preferred_element_type=jnp.float32)

    @pl.when(k == NK - 1)
    def _tail():
        a1 = acc1[...] + ftb_ref[...]
        a2 = acc2[...] + ftb_ref[...]
        f1, p1 = a1[:, :K_HALF], a1[:, K_HALF:K_HALF + 1]
        f2, p2 = a2[:, :K_HALF], a2[:, K_HALF:K_HALF + 1]
        ft_out = jnp.clip(jnp.concatenate([f1, f2], axis=1), 0.0, 1.0)
        h1 = jnp.clip(
            jax.lax.dot_general(ft_out, h1w_ref[...], _DN,
                                preferred_element_type=jnp.float32)
            + h1b_ref[...], 0.0, 1.0)
        h2 = jnp.clip(
            jax.lax.dot_general(h1, h2w_ref[...], _DN,
                                preferred_element_type=jnp.float32)
            + h2b_ref[...], 0.0, 1.0)
        o_ref[...] = (jnp.sum(h2 * outw_ref[...], axis=1, keepdims=True)
                      + outb_ref[...] + 0.125 * (p1 - p2))


def kernel(features1, features2, ft_w, ft_b, h1_w, h1_b, h2_w, h2_b,
           out_w, out_b):
    n_ft = ft_w.shape[0]
    full = lambda shape: pl.BlockSpec(shape, lambda k: tuple(0 for _ in shape))
    grid_spec = pltpu.PrefetchScalarGridSpec(
        num_scalar_prefetch=0,
        grid=(NK,),
        in_specs=[
            pl.BlockSpec((B, BK), lambda k: (0, k)),
            pl.BlockSpec((B, BK), lambda k: (0, k)),
            pl.BlockSpec((n_ft, BK), lambda k: (0, k)),
            full((1, n_ft)),
            full(h1_w.shape),
            full((1, h1_w.shape[0])),
            full(h2_w.shape),
            full((1, h2_w.shape[0])),
            full(out_w.shape),
            full((1, 1)),
        ],
        out_specs=pl.BlockSpec((B, 1), lambda k: (0, 0)),
        scratch_shapes=[
            pltpu.VMEM((B, n_ft), jnp.float32),
            pltpu.VMEM((B, n_ft), jnp.float32),
        ],
    )
    out = pl.pallas_call(
        _nnue_body,
        grid_spec=grid_spec,
        out_shape=jax.ShapeDtypeStruct((B, 1), jnp.float32),
        compiler_params=pltpu.CompilerParams(
            dimension_semantics=("arbitrary",),
        ),
    )(features1, features2, ft_w,
      ft_b.reshape(1, -1), h1_w, h1_b.reshape(1, -1),
      h2_w, h2_b.reshape(1, -1), out_w, out_b.reshape(1, 1))
    return out


# speedup vs baseline: 1.2131x; 1.2131x over previous
import jax
import jax.numpy as jnp
from jax.experimental import pallas as pl
from jax.experimental.pallas import tpu as pltpu

B = 1024
FT_IN = 41600
K_HALF = 256
BK = 2560
NK = -(-FT_IN // BK)

_DN = (((1,), (1,)), ((), ()))


def _nnue_body(x1_ref, x2_ref, w_ref, ftb_ref, h1w_ref, h1b_ref,
               h2w_ref, h2b_ref, outw_ref, outb_ref, o_ref, acc1, acc2):
    k = pl.program_id(0)

    @pl.when(k == 0)
    def _init():
        acc1[...] = jnp.zeros_like(acc1)
        acc2[...] = jnp.zeros_like(acc2)

    rem = FT_IN - k * BK
    valid = jax.lax.broadcasted_iota(jnp.int32, (1, BK), 1) < rem
    w = jnp.where(valid, w_ref[...], 0.0)
    x1 = jnp.where(valid, x1_ref[...], 0.0)
    x2 = jnp.where(valid, x2_ref[...], 0.0)
    acc1[...] += jax.lax.dot_general(x1, w, _DN,
                                     preferred_element_type=jnp.float32)
    acc2[...] += jax.lax.dot_general(x2, w, _DN,
                                     preferred_element_type=jnp.float32)

    @pl.when(k == NK - 1)
    def _tail():
        a1 = acc1[...] + ftb_ref[...]
        a2 = acc2[...] + ftb_ref[...]
        f1, p1 = a1[:, :K_HALF], a1[:, K_HALF:K_HALF + 1]
        f2, p2 = a2[:, :K_HALF], a2[:, K_HALF:K_HALF + 1]
        ft_out = jnp.clip(jnp.concatenate([f1, f2], axis=1), 0.0, 1.0)
        h1 = jnp.clip(
            jax.lax.dot_general(ft_out, h1w_ref[...], _DN,
                                preferred_element_type=jnp.float32)
            + h1b_ref[...], 0.0, 1.0)
        h2 = jnp.clip(
            jax.lax.dot_general(h1, h2w_ref[...], _DN,
                                preferred_element_type=jnp.float32)
            + h2b_ref[...], 0.0, 1.0)
        o_ref[...] = (jnp.sum(h2 * outw_ref[...], axis=1, keepdims=True)
                      + outb_ref[...] + 0.125 * (p1 - p2))


def kernel(features1, features2, ft_w, ft_b, h1_w, h1_b, h2_w, h2_b,
           out_w, out_b):
    n_ft = ft_w.shape[0]
    full = lambda shape: pl.BlockSpec(shape, lambda k: tuple(0 for _ in shape))
    grid_spec = pltpu.PrefetchScalarGridSpec(
        num_scalar_prefetch=0,
        grid=(NK,),
        in_specs=[
            pl.BlockSpec((B, BK), lambda k: (0, k)),
            pl.BlockSpec((B, BK), lambda k: (0, k)),
            pl.BlockSpec((n_ft, BK), lambda k: (0, k)),
            full((1, n_ft)),
            full(h1_w.shape),
            full((1, h1_w.shape[0])),
            full(h2_w.shape),
            full((1, h2_w.shape[0])),
            full(out_w.shape),
            full((1, 1)),
        ],
        out_specs=pl.BlockSpec((B, 1), lambda k: (0, 0)),
        scratch_shapes=[
            pltpu.VMEM((B, n_ft), jnp.float32),
            pltpu.VMEM((B, n_ft), jnp.float32),
        ],
    )
    out = pl.pallas_call(
        _nnue_body,
        grid_spec=grid_spec,
        out_shape=jax.ShapeDtypeStruct((B, 1), jnp.float32),
        compiler_params=pltpu.CompilerParams(
            dimension_semantics=("arbitrary",),
        ),
    )(features1, features2, ft_w,
      ft_b.reshape(1, -1), h1_w, h1_b.reshape(1, -1),
      h2_w, h2_b.reshape(1, -1), out_w, out_b.reshape(1, 1))
    return out
